# Initial kernel scaffold; baseline (speedup 1.0000x reference)
import jax
import jax.numpy as jnp
from jax.experimental import pallas as pl


def kernel(input_ids, table, W, b):
    raise NotImplementedError("write your pallas kernel here")


# speedup vs baseline: 4.1882x; 4.1882x over previous
import functools

import jax
import jax.numpy as jnp
from jax import lax
from jax.experimental import pallas as pl
from jax.experimental.pallas import tpu as pltpu
from jax.experimental.pallas import tpu_sc as plsc

_VOCAB = 100000
_D = 128
_NL = 16
_B = 1024
_S = 200

_NC, _NS = 2, 16
_NW = _NC * _NS
_SPW = _B // _NW
_IPW = _SPW * _S
_CHUNK = 128
_NCHUNK = _IPW // _CHUNK

_BR = 2000


def _proj_body(t_ref, w_ref, o_ref):
    o_ref[...] = lax.dot_general(
        t_ref[...], w_ref[...],
        (((1,), (1,)), ((), ())),
        preferred_element_type=jnp.float32,
    )


def _project(table, W):
    return pl.pallas_call(
        _proj_body,
        grid=(_VOCAB // _BR,),
        in_specs=[
            pl.BlockSpec((_BR, _D), lambda i: (i, 0)),
            pl.BlockSpec((_NL, _D), lambda i: (0, 0)),
        ],
        out_specs=pl.BlockSpec((_BR, _NL), lambda i: (i, 0)),
        out_shape=jax.ShapeDtypeStruct((_VOCAB, _NL), jnp.float32),
    )(table, W)


def _sc_body(p_hbm, ids_hbm, b_hbm, out_hbm, ids_v, rows_v, out_v, b_v, sem):
    wid = lax.axis_index("s") * _NC + lax.axis_index("c")
    base = wid * _IPW
    pltpu.sync_copy(ids_hbm.at[pl.ds(base, _IPW)], ids_v)
    pltpu.sync_copy(b_hbm, b_v)

    def fire(c, carry):
        pltpu.async_copy(
            p_hbm.at[ids_v.at[pl.ds(c * _CHUNK, _CHUNK)]],
            rows_v.at[pl.ds(c * _CHUNK, _CHUNK)],
            sem,
        ).wait()
        return carry

    lax.fori_loop(0, _NCHUNK, fire, 0)

    bvec = b_v[...]

    def per_sample(s, carry):
        def add(j, acc):
            return acc + rows_v[s * _S + j]

        acc = lax.fori_loop(0, _S, add, jnp.zeros((_NL,), jnp.float32))
        out_v[s] = acc * (1.0 / _S) + bvec
        return carry

    lax.fori_loop(0, _SPW, per_sample, 0)
    pltpu.sync_copy(out_v, out_hbm.at[pl.ds(wid * _SPW, _SPW)])


@functools.partial(
    pl.kernel,
    out_type=jax.ShapeDtypeStruct((_B, _NL), jnp.float32),
    mesh=plsc.VectorSubcoreMesh(core_axis_name="c", subcore_axis_name="s"),
    compiler_params=pltpu.CompilerParams(use_tc_tiling_on_sc=False),
    scratch_types=[
        pltpu.VMEM((_IPW,), jnp.int32),
        pltpu.VMEM((_IPW, _NL), jnp.float32),
        pltpu.VMEM((_SPW, _NL), jnp.float32),
        pltpu.VMEM((_NL,), jnp.float32),
        pltpu.SemaphoreType.DMA,
    ],
)
def _sc_pool(p_hbm, ids_hbm, b_hbm, out_hbm, ids_v, rows_v, out_v, b_v, sem):
    _sc_body(p_hbm, ids_hbm, b_hbm, out_hbm, ids_v, rows_v, out_v, b_v, sem)


def kernel(input_ids, table, W, b):
    P = _project(table, W)
    ids_flat = input_ids.reshape(-1).astype(jnp.int32)
    return _sc_pool(P, ids_flat, b)


# speedup vs baseline: 5.1022x; 1.2182x over previous
import functools

import jax
import jax.numpy as jnp
from jax import lax
from jax.experimental import pallas as pl
from jax.experimental.pallas import tpu as pltpu
from jax.experimental.pallas import tpu_sc as plsc

_VOCAB = 100000
_D = 128
_NL = 16
_B = 1024
_S = 200

_NC, _NS = 2, 16
_NW = _NC * _NS
_SPW = _B // _NW
_IPW = _SPW * _S
_CHUNK = 128
_NCHUNK = _IPW // _CHUNK

_BR = 2000


def _proj_body(t_ref, w_ref, o_ref):
    o_ref[...] = lax.dot_general(
        t_ref[...], w_ref[...],
        (((1,), (1,)), ((), ())),
        preferred_element_type=jnp.float32,
    )


def _project(table, W):
    return pl.pallas_call(
        _proj_body,
        grid=(_VOCAB // _BR,),
        in_specs=[
            pl.BlockSpec((_BR, _D), lambda i: (i, 0)),
            pl.BlockSpec((_NL, _D), lambda i: (0, 0)),
        ],
        out_specs=pl.BlockSpec((_BR, _NL), lambda i: (i, 0)),
        out_shape=jax.ShapeDtypeStruct((_VOCAB, _NL), jnp.float32),
    )(table, W)


def _sc_body(p_hbm, ids_hbm, b_hbm, out_hbm, ids_v, rows_v, out_v, b_v, sem):
    wid = lax.axis_index("s") * _NC + lax.axis_index("c")
    base = wid * _IPW
    pltpu.sync_copy(ids_hbm.at[pl.ds(base, _IPW)], ids_v)
    pltpu.sync_copy(b_hbm, b_v)

    def fire(c, carry):
        pltpu.async_copy(
            p_hbm.at[ids_v.at[pl.ds(c * _CHUNK, _CHUNK)]],
            rows_v.at[pl.ds(c * _CHUNK, _CHUNK)],
            sem,
        )
        return carry

    lax.fori_loop(0, _NCHUNK, fire, 0)
    pltpu.make_async_copy(p_hbm.at[pl.ds(0, _IPW)], rows_v, sem).wait()

    bvec = b_v[...]

    def per_sample(s, carry):
        def add(j, acc):
            return acc + rows_v[s * _S + j]

        acc = lax.fori_loop(0, _S, add, jnp.zeros((_NL,), jnp.float32))
        out_v[s] = acc * (1.0 / _S) + bvec
        return carry

    lax.fori_loop(0, _SPW, per_sample, 0)
    pltpu.sync_copy(out_v, out_hbm.at[pl.ds(wid * _SPW, _SPW)])


@functools.partial(
    pl.kernel,
    out_type=jax.ShapeDtypeStruct((_B, _NL), jnp.float32),
    mesh=plsc.VectorSubcoreMesh(core_axis_name="c", subcore_axis_name="s"),
    compiler_params=pltpu.CompilerParams(use_tc_tiling_on_sc=False),
    scratch_types=[
        pltpu.VMEM((_IPW,), jnp.int32),
        pltpu.VMEM((_IPW, _NL), jnp.float32),
        pltpu.VMEM((_SPW, _NL), jnp.float32),
        pltpu.VMEM((_NL,), jnp.float32),
        pltpu.SemaphoreType.DMA,
    ],
)
def _sc_pool(p_hbm, ids_hbm, b_hbm, out_hbm, ids_v, rows_v, out_v, b_v, sem):
    _sc_body(p_hbm, ids_hbm, b_hbm, out_hbm, ids_v, rows_v, out_v, b_v, sem)


def kernel(input_ids, table, W, b):
    P = _project(table, W)
    ids_flat = input_ids.reshape(-1).astype(jnp.int32)
    return _sc_pool(P, ids_flat, b)


# speedup vs baseline: 5.9870x; 1.1734x over previous
import functools

import jax
import jax.numpy as jnp
from jax import lax
from jax.experimental import pallas as pl
from jax.experimental.pallas import tpu as pltpu
from jax.experimental.pallas import tpu_sc as plsc

_VOCAB = 100000
_D = 128
_NL = 16
_B = 1024
_S = 200

_NC, _NS = 2, 16
_NW = _NC * _NS
_SPW = _B // _NW
_IPW = _SPW * _S
_CHUNK = 128
_NCHUNK = _IPW // _CHUNK

_NBLK = 20
_BR = _VOCAB // _NBLK
_NBUF = 4


def _proj_body(t_hbm, w_ref, o_hbm, tbuf, obuf, insem, outsem):
    def in_copy(i, b):
        return pltpu.make_async_copy(
            t_hbm.at[pl.ds(i * _BR, _BR), :], tbuf.at[b], insem.at[b])

    def out_copy(i, b):
        return pltpu.make_async_copy(
            obuf.at[b], o_hbm.at[pl.ds(i * _BR, _BR), :], outsem.at[b])

    for i in range(_NBUF):
        in_copy(i, i).start()
    w = w_ref[...]
    for i in range(_NBLK):
        b = i % _NBUF
        if i >= 1 and i - 1 + _NBUF < _NBLK:
            in_copy(i - 1 + _NBUF, (i - 1) % _NBUF).start()
        in_copy(i, b).wait()
        if i >= _NBUF:
            out_copy(i - _NBUF, b).wait()
        obuf[b] = lax.dot_general(
            tbuf[b], w, (((1,), (1,)), ((), ())),
            preferred_element_type=jnp.float32,
        )
        out_copy(i, b).start()
    for i in range(_NBLK - _NBUF, _NBLK):
        out_copy(i, i % _NBUF).wait()


def _project(table, W):
    return pl.pallas_call(
        _proj_body,
        in_specs=[
            pl.BlockSpec(memory_space=pl.ANY),
            pl.BlockSpec((_NL, _D), lambda: (0, 0)),
        ],
        out_specs=pl.BlockSpec(memory_space=pl.ANY),
        out_shape=jax.ShapeDtypeStruct((_VOCAB, _NL), jnp.float32),
        scratch_shapes=[
            pltpu.VMEM((_NBUF, _BR, _D), jnp.float32),
            pltpu.VMEM((_NBUF, _BR, _NL), jnp.float32),
            pltpu.SemaphoreType.DMA((_NBUF,)),
            pltpu.SemaphoreType.DMA((_NBUF,)),
        ],
    )(table, W)


def _sc_body(p_hbm, ids_hbm, b_hbm, out_hbm, ids_v, rows_v, out_v, b_v, sem):
    wid = lax.axis_index("s") * _NC + lax.axis_index("c")
    base = wid * _IPW
    pltpu.sync_copy(ids_hbm.at[pl.ds(base, _IPW)], ids_v)
    pltpu.sync_copy(b_hbm, b_v)

    def fire(c, carry):
        pltpu.async_copy(
            p_hbm.at[ids_v.at[pl.ds(c * _CHUNK, _CHUNK)]],
            rows_v.at[pl.ds(c * _CHUNK, _CHUNK)],
            sem,
        )
        return carry

    lax.fori_loop(0, _NCHUNK, fire, 0)
    pltpu.make_async_copy(p_hbm.at[pl.ds(0, _IPW)], rows_v, sem).wait()

    bvec = b_v[...]

    def per_sample(s, carry):
        def add(j, acc):
            return acc + rows_v[s * _S + j]

        acc = lax.fori_loop(0, _S, add, jnp.zeros((_NL,), jnp.float32))
        out_v[s] = acc * (1.0 / _S) + bvec
        return carry

    lax.fori_loop(0, _SPW, per_sample, 0)
    pltpu.sync_copy(out_v, out_hbm.at[pl.ds(wid * _SPW, _SPW)])


@functools.partial(
    pl.kernel,
    out_type=jax.ShapeDtypeStruct((_B, _NL), jnp.float32),
    mesh=plsc.VectorSubcoreMesh(core_axis_name="c", subcore_axis_name="s"),
    compiler_params=pltpu.CompilerParams(use_tc_tiling_on_sc=False),
    scratch_types=[
        pltpu.VMEM((_IPW,), jnp.int32),
        pltpu.VMEM((_IPW, _NL), jnp.float32),
        pltpu.VMEM((_SPW, _NL), jnp.float32),
        pltpu.VMEM((_NL,), jnp.float32),
        pltpu.SemaphoreType.DMA,
    ],
)
def _sc_pool(p_hbm, ids_hbm, b_hbm, out_hbm, ids_v, rows_v, out_v, b_v, sem):
    _sc_body(p_hbm, ids_hbm, b_hbm, out_hbm, ids_v, rows_v, out_v, b_v, sem)


def kernel(input_ids, table, W, b):
    P = _project(table, W)
    ids_flat = input_ids.reshape(-1).astype(jnp.int32)
    return _sc_pool(P, ids_flat, b)
